# baseline (speedup 1.0000x reference)
import dataclasses

import jax
import jax.numpy as jnp
from jax import lax
from jax.experimental import pallas as pl
from jax.experimental.pallas import tpu as pltpu
from jax.experimental.pallas import tpu_sc as plsc

B = 16384
F = 64
L = 16
NC, NS = 2, 16
NW = NC * NS
BPW = B // NW
SR = 8
W = 2 * F
RPT = 16
CH = 32
NCHUNK = BPW // CH


def _cf_body(users_hbm, items_hbm, ue_hbm, ub_hbm, ie_hbm, ib_hbm, out_hbm,
             uidx_v, iidx_v, ugidx_v, igidx_v, ue_v, ie_v, ub_v, ib_v, out_v,
             sem_u, sem_i, sem_ub, sem_ib):
    wid = lax.axis_index("s") * NC + lax.axis_index("c")
    base = wid * BPW

    pltpu.sync_copy(users_hbm.at[pl.ds(base, BPW)], uidx_v)
    pltpu.sync_copy(items_hbm.at[pl.ds(base, BPW)], iidx_v)

    @pl.loop(0, BPW, step=L)
    def _(k):
        ugidx_v[pl.ds(k, L)] = uidx_v[pl.ds(k, L)] >> 4
        igidx_v[pl.ds(k, L)] = iidx_v[pl.ds(k, L)] >> 4

    cub = pltpu.async_copy(ub_hbm.at[uidx_v], ub_v, sem_ub)
    cib = pltpu.async_copy(ib_hbm.at[iidx_v], ib_v, sem_ib)

    lane = lax.broadcasted_iota(jnp.int32, (L,), 0)
    nc = F // L

    for t in range(NCHUNK):
        cu = pltpu.async_copy(ue_hbm.at[ugidx_v.at[pl.ds(t * CH, CH)]],
                              ue_v, sem_u)
        ci = pltpu.async_copy(ie_hbm.at[igidx_v.at[pl.ds(t * CH, CH)]],
                              ie_v, sem_i)
        cu.wait()
        ci.wait()

        @pl.loop(0, CH, step=L)
        def _(g):
            gg = t * CH + g
            uvec = uidx_v[pl.ds(gg, L)]
            ivec = iidx_v[pl.ds(gg, L)]
            usub = (uvec >> 1) & (SR - 1)
            isub = (ivec >> 1) & (SR - 1)
            uoff = (uvec & 1) * F
            ioff = (ivec & 1) * F

            def row(j, res):
                b = g + j
                su, ou = usub[j], uoff[j]
                si, oi = isub[j], ioff[j]
                acc = (ue_v[b, su, pl.ds(ou, L)]
                       * ie_v[b, si, pl.ds(oi, L)])
                for c in range(1, nc):
                    acc = acc + (ue_v[b, su, pl.ds(ou + c * L, L)]
                                 * ie_v[b, si, pl.ds(oi + c * L, L)])
                return jnp.where(lane == j, jnp.sum(acc), res)

            res = jnp.zeros((L,), jnp.float32)
            for j in range(L):
                res = row(j, res)
            out_v[pl.ds(gg, L)] = res

    cub.wait()
    cib.wait()

    @pl.loop(0, BPW, step=L)
    def _(g):
        out_v[pl.ds(g, L)] = (out_v[pl.ds(g, L)] + ub_v[pl.ds(g, L)]
                              + ib_v[pl.ds(g, L)])

    pltpu.sync_copy(out_v, out_hbm.at[pl.ds(base, BPW)])


def kernel(users, items, user_emb, user_bias, item_emb, item_bias):
    mesh = plsc.VectorSubcoreMesh(core_axis_name="c", subcore_axis_name="s")
    cp = pltpu.CompilerParams()
    if "needs_layout_passes" in pltpu.CompilerParams.__dataclass_fields__:
        cp = dataclasses.replace(cp, needs_layout_passes=False)
    k = pl.kernel(
        _cf_body,
        out_type=jax.ShapeDtypeStruct((B,), jnp.float32),
        mesh=mesh,
        compiler_params=cp,
        scratch_types=[
            pltpu.VMEM((BPW,), jnp.int32),
            pltpu.VMEM((BPW,), jnp.int32),
            pltpu.VMEM((BPW,), jnp.int32),
            pltpu.VMEM((BPW,), jnp.int32),
            pltpu.VMEM((CH, SR, W), jnp.float32),
            pltpu.VMEM((CH, SR, W), jnp.float32),
            pltpu.VMEM((BPW,), jnp.float32),
            pltpu.VMEM((BPW,), jnp.float32),
            pltpu.VMEM((BPW,), jnp.float32),
            pltpu.SemaphoreType.DMA,
            pltpu.SemaphoreType.DMA,
            pltpu.SemaphoreType.DMA,
            pltpu.SemaphoreType.DMA,
        ],
    )
    n_users = user_emb.shape[0]
    n_items = item_emb.shape[0]
    return k(users.astype(jnp.int32), items.astype(jnp.int32),
             user_emb.reshape(n_users // RPT, SR, W), user_bias.reshape(-1),
             item_emb.reshape(n_items // RPT, SR, W), item_bias.reshape(-1))


# speedup vs baseline: 1.1167x; 1.1167x over previous
import dataclasses

import jax
import jax.numpy as jnp
from jax import lax
from jax.experimental import pallas as pl
from jax.experimental.pallas import tpu as pltpu
from jax.experimental.pallas import tpu_sc as plsc

B = 16384
F = 64
L = 16
NC, NS = 2, 16
NW = NC * NS
BPW = B // NW
W = 2 * F
CH = 128
NCHUNK = BPW // CH


def _cf_body(users_hbm, items_hbm, ue_hbm, ub_hbm, ie_hbm, ib_hbm, out_hbm,
             uidx_v, iidx_v, ue_v, ie_v, ub_v, ib_v, out_v,
             sem_u, sem_i, sem_ub, sem_ib):
    wid = lax.axis_index("s") * NC + lax.axis_index("c")
    base = wid * BPW

    pltpu.sync_copy(users_hbm.at[pl.ds(base, BPW)], uidx_v)
    pltpu.sync_copy(items_hbm.at[pl.ds(base, BPW)], iidx_v)

    cub = pltpu.async_copy(ub_hbm.at[uidx_v], ub_v, sem_ub)
    cib = pltpu.async_copy(ib_hbm.at[iidx_v], ib_v, sem_ib)

    lane = lax.broadcasted_iota(jnp.int32, (L,), 0)
    nc = F // L

    for t in range(NCHUNK):
        cu = pltpu.async_copy(ue_hbm.at[uidx_v.at[pl.ds(t * CH, CH)]],
                              ue_v, sem_u)
        ci = pltpu.async_copy(ie_hbm.at[iidx_v.at[pl.ds(t * CH, CH)]],
                              ie_v, sem_i)
        cu.wait()
        ci.wait()

        @pl.loop(0, CH, step=L)
        def _(g):
            gg = t * CH + g

            def row(j, res):
                b = g + j
                acc = ue_v[b, pl.ds(0, L)] * ie_v[b, pl.ds(0, L)]
                for c in range(1, nc):
                    acc = acc + (ue_v[b, pl.ds(c * L, L)]
                                 * ie_v[b, pl.ds(c * L, L)])
                return jnp.where(lane == j, jnp.sum(acc), res)

            res = jnp.zeros((L,), jnp.float32)
            for j in range(L):
                res = row(j, res)
            out_v[pl.ds(gg, L)] = res

    cub.wait()
    cib.wait()

    @pl.loop(0, BPW, step=L)
    def _(g):
        out_v[pl.ds(g, L)] = (out_v[pl.ds(g, L)] + ub_v[pl.ds(g, L)]
                              + ib_v[pl.ds(g, L)])

    pltpu.sync_copy(out_v, out_hbm.at[pl.ds(base, BPW)])


def kernel(users, items, user_emb, user_bias, item_emb, item_bias):
    mesh = plsc.VectorSubcoreMesh(core_axis_name="c", subcore_axis_name="s")
    cp = pltpu.CompilerParams()
    if "needs_layout_passes" in pltpu.CompilerParams.__dataclass_fields__:
        cp = dataclasses.replace(cp, needs_layout_passes=False)
    k = pl.kernel(
        _cf_body,
        out_type=jax.ShapeDtypeStruct((B,), jnp.float32),
        mesh=mesh,
        compiler_params=cp,
        scratch_types=[
            pltpu.VMEM((BPW,), jnp.int32),
            pltpu.VMEM((BPW,), jnp.int32),
            pltpu.VMEM((CH, W), jnp.float32),
            pltpu.VMEM((CH, W), jnp.float32),
            pltpu.VMEM((BPW,), jnp.float32),
            pltpu.VMEM((BPW,), jnp.float32),
            pltpu.VMEM((BPW,), jnp.float32),
            pltpu.SemaphoreType.DMA,
            pltpu.SemaphoreType.DMA,
            pltpu.SemaphoreType.DMA,
            pltpu.SemaphoreType.DMA,
        ],
    )
    pad = ((0, 0), (0, W - F))
    return k(users.astype(jnp.int32), items.astype(jnp.int32),
             jnp.pad(user_emb, pad), user_bias.reshape(-1),
             jnp.pad(item_emb, pad), item_bias.reshape(-1))


# speedup vs baseline: 1.2675x; 1.1351x over previous
import dataclasses

import jax
import jax.numpy as jnp
from jax import lax
from jax.experimental import pallas as pl
from jax.experimental.pallas import tpu as pltpu
from jax.experimental.pallas import tpu_sc as plsc

B = 16384
F = 64
L = 16
NC, NS = 2, 16
NW = NC * NS
BPW = B // NW
W = 2 * F
CH = 128
NCHUNK = BPW // CH


def _cf_body(users_hbm, items_hbm, tab_hbm, ub_hbm, ib_hbm, out_hbm,
             uidx_v, iidx_v, ue_v, ie_v, ub_v, ib_v, out_v,
             sem_u, sem_i, sem_ub, sem_ib):
    wid = lax.axis_index("s") * NC + lax.axis_index("c")
    base = wid * BPW

    pltpu.sync_copy(users_hbm.at[pl.ds(base, BPW)], uidx_v)
    pltpu.sync_copy(items_hbm.at[pl.ds(base, BPW)], iidx_v)

    cub = pltpu.async_copy(ub_hbm.at[uidx_v], ub_v, sem_ub)
    cib = pltpu.async_copy(ib_hbm.at[iidx_v], ib_v, sem_ib)

    lane = lax.broadcasted_iota(jnp.int32, (L,), 0)
    nc = F // L

    for t in range(NCHUNK):
        cu = pltpu.async_copy(tab_hbm.at[uidx_v.at[pl.ds(t * CH, CH)]],
                              ue_v, sem_u)
        ci = pltpu.async_copy(tab_hbm.at[iidx_v.at[pl.ds(t * CH, CH)]],
                              ie_v, sem_i)
        cu.wait()
        ci.wait()

        @pl.loop(0, CH, step=L)
        def _(g):
            gg = t * CH + g

            def row(j, res):
                b = g + j
                acc = ue_v[b, pl.ds(0, L)] * ie_v[b, pl.ds(F, L)]
                for c in range(1, nc):
                    acc = acc + (ue_v[b, pl.ds(c * L, L)]
                                 * ie_v[b, pl.ds(F + c * L, L)])
                return jnp.where(lane == j, jnp.sum(acc), res)

            res = jnp.zeros((L,), jnp.float32)
            for j in range(L):
                res = row(j, res)
            out_v[pl.ds(gg, L)] = res

    cub.wait()
    cib.wait()

    @pl.loop(0, BPW, step=L)
    def _(g):
        out_v[pl.ds(g, L)] = (out_v[pl.ds(g, L)] + ub_v[pl.ds(g, L)]
                              + ib_v[pl.ds(g, L)])

    pltpu.sync_copy(out_v, out_hbm.at[pl.ds(base, BPW)])


def kernel(users, items, user_emb, user_bias, item_emb, item_bias):
    mesh = plsc.VectorSubcoreMesh(core_axis_name="c", subcore_axis_name="s")
    cp = pltpu.CompilerParams()
    if "needs_layout_passes" in pltpu.CompilerParams.__dataclass_fields__:
        cp = dataclasses.replace(cp, needs_layout_passes=False)
    k = pl.kernel(
        _cf_body,
        out_type=jax.ShapeDtypeStruct((B,), jnp.float32),
        mesh=mesh,
        compiler_params=cp,
        scratch_types=[
            pltpu.VMEM((BPW,), jnp.int32),
            pltpu.VMEM((BPW,), jnp.int32),
            pltpu.VMEM((CH, W), jnp.float32),
            pltpu.VMEM((CH, W), jnp.float32),
            pltpu.VMEM((BPW,), jnp.float32),
            pltpu.VMEM((BPW,), jnp.float32),
            pltpu.VMEM((BPW,), jnp.float32),
            pltpu.SemaphoreType.DMA,
            pltpu.SemaphoreType.DMA,
            pltpu.SemaphoreType.DMA,
            pltpu.SemaphoreType.DMA,
        ],
    )
    tab = jnp.concatenate([user_emb, item_emb], axis=1)
    return k(users.astype(jnp.int32), items.astype(jnp.int32),
             tab, user_bias.reshape(-1), item_bias.reshape(-1))
